# Initial kernel scaffold; baseline (speedup 1.0000x reference)
import jax
import jax.numpy as jnp
from jax.experimental import pallas as pl


def kernel(x, edge_index, W_enc, b_enc, W_dec, b_dec):
    raise NotImplementedError("write your pallas kernel here")


# speedup vs baseline: 19.6534x; 19.6534x over previous
import functools

import jax
import jax.numpy as jnp
from jax import lax
from jax.experimental import pallas as pl
from jax.experimental.pallas import tpu as pltpu
from jax.experimental.pallas import tpu_sc as plsc

N = 10000
D_IN = 128
D_HID = 64
E = 320000

NC = 2
NS = 16
K = 128
CHUNKS = 80
E_PAD = NC * NS * CHUNKS * K
NPAD = 10240
STRIPE = NPAD // NS
DUMP = 10200
CW = 16

def _sc_count_body(dst_hbm, out_hbm, dst_v, ones_v, acc_sh):
    c = lax.axis_index("c")
    s = lax.axis_index("s")
    base = s * STRIPE

    def _zero_row(i, carry):
        ones_v[i, :] = jnp.zeros((CW,), jnp.float32)
        return carry

    lax.fori_loop(0, K, _zero_row, 0)

    def _zero_stripe(i, carry):
        pltpu.sync_copy(ones_v, acc_sh.at[pl.ds(base + i * K, K)])
        return carry

    lax.fori_loop(0, STRIPE // K, _zero_stripe, 0)

    def _fill_row(i, carry):
        ones_v[i, :] = jnp.ones((CW,), jnp.float32)
        return carry

    lax.fori_loop(0, K, _fill_row, 0)

    pltpu.sync_copy(dst_hbm.at[c, s], dst_v)
    plsc.subcore_barrier()

    def _scatter(j, carry):
        pltpu.sync_copy(ones_v, acc_sh.at[dst_v.at[j]], add=True)
        return carry

    lax.fori_loop(0, CHUNKS, _scatter, 0)
    plsc.subcore_barrier()

    pltpu.sync_copy(acc_sh.at[pl.ds(base, STRIPE)], out_hbm.at[c, pl.ds(base, STRIPE)])


def _sc_aggregate_body(
    src_hbm, dst_hbm, g_hbm, out_hbm, src_v, dst_v, rows_v, acc_sh, sem
):
    c = lax.axis_index("c")
    s = lax.axis_index("s")
    base = s * STRIPE

    def _zero_row(i, carry):
        for kk in range(D_HID // 16):
            rows_v[i, pl.ds(kk * 16, 16)] = jnp.zeros((16,), jnp.float32)
        return carry

    lax.fori_loop(0, K, _zero_row, 0)

    def _zero_stripe(i, carry):
        pltpu.sync_copy(rows_v, acc_sh.at[pl.ds(base + i * K, K)])
        return carry

    lax.fori_loop(0, STRIPE // K, _zero_stripe, 0)

    pltpu.sync_copy(src_hbm.at[c, s], src_v)
    pltpu.sync_copy(dst_hbm.at[c, s], dst_v)
    plsc.subcore_barrier()

    def _edge_chunk(j, carry):
        pltpu.async_copy(g_hbm.at[src_v.at[j]], rows_v, sem).wait()
        pltpu.sync_copy(rows_v, acc_sh.at[dst_v.at[j]], add=True)
        return carry

    lax.fori_loop(0, CHUNKS, _edge_chunk, 0)
    plsc.subcore_barrier()

    pltpu.sync_copy(acc_sh.at[pl.ds(base, STRIPE)], out_hbm.at[c, pl.ds(base, STRIPE)])


@functools.lru_cache(maxsize=None)
def _sc_kernels():
    mesh = plsc.VectorSubcoreMesh(
        core_axis_name="c", subcore_axis_name="s", num_cores=NC, num_subcores=NS
    )
    count = pl.kernel(
        _sc_count_body,
        out_type=jax.ShapeDtypeStruct((NC, NPAD, CW), jnp.float32),
        mesh=mesh,
        scratch_types=[
            pltpu.VMEM((CHUNKS, K), jnp.int32),
            pltpu.VMEM((K, CW), jnp.float32),
            pltpu.VMEM_SHARED((NPAD, CW), jnp.float32),
        ],
        name="sc_degree_histogram",
    )
    aggregate = pl.kernel(
        _sc_aggregate_body,
        out_type=jax.ShapeDtypeStruct((NC, NPAD, D_HID), jnp.float32),
        mesh=mesh,
        scratch_types=[
            pltpu.VMEM((CHUNKS, K), jnp.int32),
            pltpu.VMEM((CHUNKS, K), jnp.int32),
            pltpu.VMEM((K, D_HID), jnp.float32),
            pltpu.VMEM_SHARED((NPAD, D_HID), jnp.float32),
            pltpu.SemaphoreType.DMA,
        ],
        name="sc_edge_aggregate",
        compiler_params=pltpu.CompilerParams(use_tc_tiling_on_sc=False),
    )
    return count, aggregate


def _enc_body(x_ref, w_ref, c0_ref, c1_ref, h_ref, g_ref, dis_ref):
    deg = c0_ref[:, 0:1] + c1_ref[:, 0:1] + 1.0
    dis = lax.rsqrt(deg)
    h = jnp.dot(x_ref[...], w_ref[...], preferred_element_type=jnp.float32)
    h_ref[...] = h
    g_ref[...] = h * dis
    dis_ref[...] = dis


_enc_call = pl.pallas_call(
    _enc_body,
    out_shape=(
        jax.ShapeDtypeStruct((N, D_HID), jnp.float32),
        jax.ShapeDtypeStruct((N, D_HID), jnp.float32),
        jax.ShapeDtypeStruct((N, 1), jnp.float32),
    ),
)


def _dec_body(a0_ref, a1_ref, g_ref, dis_ref, be_ref, wd_ref, bd_ref, z_ref, xr_ref):
    agg = a0_ref[...] + a1_ref[...] + g_ref[...]
    pre = agg * dis_ref[...] + be_ref[...]
    z = jnp.maximum(pre, 0.0)
    z_ref[...] = z
    xr_ref[...] = (
        jnp.dot(z, wd_ref[...], preferred_element_type=jnp.float32) + bd_ref[...]
    )


_dec_call = pl.pallas_call(
    _dec_body,
    out_shape=(
        jax.ShapeDtypeStruct((N, D_HID), jnp.float32),
        jax.ShapeDtypeStruct((N, D_IN), jnp.float32),
    ),
)


def kernel(x, edge_index, W_enc, b_enc, W_dec, b_dec):
    src = edge_index[0].astype(jnp.int32)
    dst = edge_index[1].astype(jnp.int32)
    pad = jnp.full((E_PAD - E,), DUMP, jnp.int32)
    src_p = jnp.concatenate([src, pad]).reshape(NC, NS, CHUNKS, K)
    dst_p = jnp.concatenate([dst, pad]).reshape(NC, NS, CHUNKS, K)

    sc_count, sc_aggregate = _sc_kernels()
    counts = sc_count(dst_p)
    h, g, dis = _enc_call(x, W_enc, counts[0, :N], counts[1, :N])
    g_pad = jnp.pad(g, ((0, NPAD - N), (0, 0)))
    agg = sc_aggregate(src_p, dst_p, g_pad)
    z, xr = _dec_call(
        agg[0, :N],
        agg[1, :N],
        g,
        dis,
        b_enc.reshape(1, D_HID),
        W_dec,
        b_dec.reshape(1, D_IN),
    )
    return (z, xr)


# speedup vs baseline: 38.1868x; 1.9430x over previous
import functools

import jax
import jax.numpy as jnp
from jax import lax
from jax.experimental import pallas as pl
from jax.experimental.pallas import tpu as pltpu
from jax.experimental.pallas import tpu_sc as plsc

N = 10000
D_IN = 128
D_HID = 64
E = 320000

NC = 2
NS = 16
K = 125
CHUNKS = 80
NPAD = 10240
STRIPE = NPAD // NS
ZB = 80
CW = 16


def _sc_count_body(dst_hbm, out_hbm, dst_v, ones_v, acc_sh):
    c = lax.axis_index("c")
    s = lax.axis_index("s")
    base = s * STRIPE

    def _zero_row(i, carry):
        ones_v[i, :] = jnp.zeros((CW,), jnp.float32)
        return carry

    lax.fori_loop(0, K, _zero_row, 0)

    def _zero_stripe(i, carry):
        pltpu.sync_copy(ones_v.at[pl.ds(0, ZB)], acc_sh.at[pl.ds(base + i * ZB, ZB)])
        return carry

    lax.fori_loop(0, STRIPE // ZB, _zero_stripe, 0)

    def _fill_row(i, carry):
        ones_v[i, :] = jnp.ones((CW,), jnp.float32)
        return carry

    lax.fori_loop(0, K, _fill_row, 0)

    pltpu.sync_copy(dst_hbm.at[c, s], dst_v)
    plsc.subcore_barrier()

    def _scatter(j, carry):
        pltpu.sync_copy(ones_v, acc_sh.at[dst_v.at[j]], add=True)
        return carry

    lax.fori_loop(0, CHUNKS, _scatter, 0)
    plsc.subcore_barrier()

    pltpu.sync_copy(acc_sh.at[pl.ds(base, STRIPE)], out_hbm.at[c, pl.ds(base, STRIPE)])


def _sc_aggregate_body(
    src_hbm, dst_hbm, g_hbm, out_hbm, src_v, dst_v, rows_a, rows_b, acc_sh, sem_a, sem_b
):
    c = lax.axis_index("c")
    s = lax.axis_index("s")
    base = s * STRIPE

    def _zero_row(i, carry):
        for kk in range(D_HID // 16):
            rows_a[i, pl.ds(kk * 16, 16)] = jnp.zeros((16,), jnp.float32)
        return carry

    lax.fori_loop(0, K, _zero_row, 0)

    def _zero_stripe(i, carry):
        pltpu.sync_copy(rows_a.at[pl.ds(0, ZB)], acc_sh.at[pl.ds(base + i * ZB, ZB)])
        return carry

    lax.fori_loop(0, STRIPE // ZB, _zero_stripe, 0)

    pltpu.sync_copy(src_hbm.at[c, s], src_v)
    pltpu.sync_copy(dst_hbm.at[c, s], dst_v)
    plsc.subcore_barrier()

    pltpu.async_copy(g_hbm.at[src_v.at[0]], rows_a, sem_a)

    def _pair(p, carry):
        j0 = 2 * p
        pltpu.make_async_copy(g_hbm.at[src_v.at[j0]], rows_a, sem_a).wait()
        pltpu.async_copy(g_hbm.at[src_v.at[j0 + 1]], rows_b, sem_b)
        pltpu.sync_copy(rows_a, acc_sh.at[dst_v.at[j0]], add=True)
        pltpu.make_async_copy(g_hbm.at[src_v.at[j0 + 1]], rows_b, sem_b).wait()

        @pl.when(p < CHUNKS // 2 - 1)
        def _():
            pltpu.async_copy(g_hbm.at[src_v.at[j0 + 2]], rows_a, sem_a)

        pltpu.sync_copy(rows_b, acc_sh.at[dst_v.at[j0 + 1]], add=True)
        return carry

    lax.fori_loop(0, CHUNKS // 2, _pair, 0)
    plsc.subcore_barrier()

    pltpu.sync_copy(acc_sh.at[pl.ds(base, STRIPE)], out_hbm.at[c, pl.ds(base, STRIPE)])


@functools.lru_cache(maxsize=None)
def _sc_kernels():
    mesh = plsc.VectorSubcoreMesh(
        core_axis_name="c", subcore_axis_name="s", num_cores=NC, num_subcores=NS
    )
    count = pl.kernel(
        _sc_count_body,
        out_type=jax.ShapeDtypeStruct((NC, NPAD, CW), jnp.float32),
        mesh=mesh,
        scratch_types=[
            pltpu.VMEM((CHUNKS, K), jnp.int32),
            pltpu.VMEM((K, CW), jnp.float32),
            pltpu.VMEM_SHARED((NPAD, CW), jnp.float32),
        ],
        name="sc_degree_histogram",
    )
    aggregate = pl.kernel(
        _sc_aggregate_body,
        out_type=jax.ShapeDtypeStruct((NC, NPAD, D_HID), jnp.float32),
        mesh=mesh,
        scratch_types=[
            pltpu.VMEM((CHUNKS, K), jnp.int32),
            pltpu.VMEM((CHUNKS, K), jnp.int32),
            pltpu.VMEM((K, D_HID), jnp.float32),
            pltpu.VMEM((K, D_HID), jnp.float32),
            pltpu.VMEM_SHARED((NPAD, D_HID), jnp.float32),
            pltpu.SemaphoreType.DMA,
            pltpu.SemaphoreType.DMA,
        ],
        name="sc_edge_aggregate",
        compiler_params=pltpu.CompilerParams(use_tc_tiling_on_sc=False),
    )
    return count, aggregate


def _enc_body(x_ref, w_ref, c0_ref, c1_ref, h_ref, g_ref, dis_ref):
    deg = c0_ref[:, 0:1] + c1_ref[:, 0:1] + 1.0
    dis = lax.rsqrt(deg)
    h = jnp.dot(x_ref[...], w_ref[...], preferred_element_type=jnp.float32)
    h_ref[...] = h
    g_ref[...] = h * dis
    dis_ref[...] = dis


_enc_call = pl.pallas_call(
    _enc_body,
    out_shape=(
        jax.ShapeDtypeStruct((N, D_HID), jnp.float32),
        jax.ShapeDtypeStruct((N, D_HID), jnp.float32),
        jax.ShapeDtypeStruct((N, 1), jnp.float32),
    ),
)


def _dec_body(a0_ref, a1_ref, g_ref, dis_ref, be_ref, wd_ref, bd_ref, z_ref, xr_ref):
    agg = a0_ref[...] + a1_ref[...] + g_ref[...]
    pre = agg * dis_ref[...] + be_ref[...]
    z = jnp.maximum(pre, 0.0)
    z_ref[...] = z
    xr_ref[...] = (
        jnp.dot(z, wd_ref[...], preferred_element_type=jnp.float32) + bd_ref[...]
    )


_dec_call = pl.pallas_call(
    _dec_body,
    out_shape=(
        jax.ShapeDtypeStruct((N, D_HID), jnp.float32),
        jax.ShapeDtypeStruct((N, D_IN), jnp.float32),
    ),
)


def kernel(x, edge_index, W_enc, b_enc, W_dec, b_dec):
    src_p = edge_index[0].astype(jnp.int32).reshape(NC, NS, CHUNKS, K)
    dst_p = edge_index[1].astype(jnp.int32).reshape(NC, NS, CHUNKS, K)

    sc_count, sc_aggregate = _sc_kernels()
    counts = sc_count(dst_p)
    h, g, dis = _enc_call(x, W_enc, counts[0, :N], counts[1, :N])
    agg = sc_aggregate(src_p, dst_p, g)
    z, xr = _dec_call(
        agg[0, :N],
        agg[1, :N],
        g,
        dis,
        b_enc.reshape(1, D_HID),
        W_dec,
        b_dec.reshape(1, D_IN),
    )
    return (z, xr)


# speedup vs baseline: 45.3445x; 1.1874x over previous
import functools

import jax
import jax.numpy as jnp
from jax import lax
from jax.experimental import pallas as pl
from jax.experimental.pallas import tpu as pltpu
from jax.experimental.pallas import tpu_sc as plsc

N = 10000
D_IN = 128
D_HID = 64
E = 320000

NC = 2
NS = 16
K = 125
CHUNKS = 80
NPAD = 10240
STRIPE = NPAD // NS
ZB = 80
CW = 16


def _sc_count_body(dst_hbm, out_hbm, dst_v, ones_v, acc_sh, sem):
    c = lax.axis_index("c")
    s = lax.axis_index("s")
    base = s * STRIPE

    def _zero_row(i, carry):
        ones_v[i, :] = jnp.zeros((CW,), jnp.float32)
        return carry

    lax.fori_loop(0, K, _zero_row, 0)

    def _zero_stripe(i, carry):
        pltpu.sync_copy(ones_v.at[pl.ds(0, ZB)], acc_sh.at[pl.ds(base + i * ZB, ZB)])
        return carry

    lax.fori_loop(0, STRIPE // ZB, _zero_stripe, 0)

    def _fill_row(i, carry):
        ones_v[i, :] = jnp.ones((CW,), jnp.float32)
        return carry

    lax.fori_loop(0, K, _fill_row, 0)

    pltpu.sync_copy(dst_hbm.at[c, s], dst_v)
    plsc.subcore_barrier()

    def _scatter(j, carry):
        pltpu.async_copy(ones_v, acc_sh.at[dst_v.at[j]], sem, add=True)
        return carry

    lax.fori_loop(0, CHUNKS, _scatter, 0)

    def _drain(j, carry):
        pltpu.make_async_copy(ones_v, acc_sh.at[dst_v.at[0]], sem).wait()
        return carry

    lax.fori_loop(0, CHUNKS, _drain, 0)
    plsc.subcore_barrier()

    pltpu.sync_copy(acc_sh.at[pl.ds(base, STRIPE)], out_hbm.at[c, pl.ds(base, STRIPE)])


NBUF = 4
ROUNDS = CHUNKS // NBUF


def _sc_aggregate_body(src_hbm, dst_hbm, g_hbm, out_hbm, src_v, dst_v, *rest):
    rows = rest[:NBUF]
    acc_sh = rest[NBUF]
    sem_g = rest[NBUF + 1 : NBUF + 1 + NBUF]
    sem_s = rest[NBUF + 1 + NBUF :]
    c = lax.axis_index("c")
    s = lax.axis_index("s")
    base = s * STRIPE

    def _zero_row(i, carry):
        for kk in range(D_HID // 16):
            rows[0][i, pl.ds(kk * 16, 16)] = jnp.zeros((16,), jnp.float32)
        return carry

    lax.fori_loop(0, K, _zero_row, 0)

    def _zero_stripe(i, carry):
        pltpu.sync_copy(rows[0].at[pl.ds(0, ZB)], acc_sh.at[pl.ds(base + i * ZB, ZB)])
        return carry

    lax.fori_loop(0, STRIPE // ZB, _zero_stripe, 0)

    pltpu.sync_copy(src_hbm.at[c, s], src_v)
    pltpu.sync_copy(dst_hbm.at[c, s], dst_v)
    plsc.subcore_barrier()

    for b in range(NBUF):
        pltpu.async_copy(g_hbm.at[src_v.at[b]], rows[b], sem_g[b])

    def _round(r, carry):
        j0 = r * NBUF
        for b in range(NBUF):
            pltpu.make_async_copy(g_hbm.at[src_v.at[j0 + b]], rows[b], sem_g[b]).wait()
            pltpu.async_copy(rows[b], acc_sh.at[dst_v.at[j0 + b]], sem_s[b], add=True)
        for b in range(NBUF):
            pltpu.make_async_copy(rows[b], acc_sh.at[dst_v.at[0]], sem_s[b]).wait()

            @pl.when(r < ROUNDS - 1)
            def _():
                pltpu.async_copy(g_hbm.at[src_v.at[j0 + NBUF + b]], rows[b], sem_g[b])

        return carry

    lax.fori_loop(0, ROUNDS, _round, 0)
    plsc.subcore_barrier()

    pltpu.sync_copy(acc_sh.at[pl.ds(base, STRIPE)], out_hbm.at[c, pl.ds(base, STRIPE)])


@functools.lru_cache(maxsize=None)
def _sc_kernels():
    mesh = plsc.VectorSubcoreMesh(
        core_axis_name="c", subcore_axis_name="s", num_cores=NC, num_subcores=NS
    )
    count = pl.kernel(
        _sc_count_body,
        out_type=jax.ShapeDtypeStruct((NC, NPAD, CW), jnp.float32),
        mesh=mesh,
        scratch_types=[
            pltpu.VMEM((CHUNKS, K), jnp.int32),
            pltpu.VMEM((K, CW), jnp.float32),
            pltpu.VMEM_SHARED((NPAD, CW), jnp.float32),
            pltpu.SemaphoreType.DMA,
        ],
        name="sc_degree_histogram",
    )
    aggregate = pl.kernel(
        _sc_aggregate_body,
        out_type=jax.ShapeDtypeStruct((NC, NPAD, D_HID), jnp.float32),
        mesh=mesh,
        scratch_types=[
            pltpu.VMEM((CHUNKS, K), jnp.int32),
            pltpu.VMEM((CHUNKS, K), jnp.int32),
        ]
        + [pltpu.VMEM((K, D_HID), jnp.float32) for _ in range(NBUF)]
        + [pltpu.VMEM_SHARED((NPAD, D_HID), jnp.float32)]
        + [pltpu.SemaphoreType.DMA for _ in range(2 * NBUF)],
        name="sc_edge_aggregate",
        compiler_params=pltpu.CompilerParams(use_tc_tiling_on_sc=False),
    )
    return count, aggregate


def _enc_body(x_ref, w_ref, c0_ref, c1_ref, h_ref, g_ref, dis_ref):
    deg = c0_ref[:, 0:1] + c1_ref[:, 0:1] + 1.0
    dis = lax.rsqrt(deg)
    h = jnp.dot(x_ref[...], w_ref[...], preferred_element_type=jnp.float32)
    h_ref[...] = h
    g_ref[...] = h * dis
    dis_ref[...] = dis


_enc_call = pl.pallas_call(
    _enc_body,
    out_shape=(
        jax.ShapeDtypeStruct((N, D_HID), jnp.float32),
        jax.ShapeDtypeStruct((N, D_HID), jnp.float32),
        jax.ShapeDtypeStruct((N, 1), jnp.float32),
    ),
)


def _dec_body(a0_ref, a1_ref, g_ref, dis_ref, be_ref, wd_ref, bd_ref, z_ref, xr_ref):
    agg = a0_ref[...] + a1_ref[...] + g_ref[...]
    pre = agg * dis_ref[...] + be_ref[...]
    z = jnp.maximum(pre, 0.0)
    z_ref[...] = z
    xr_ref[...] = (
        jnp.dot(z, wd_ref[...], preferred_element_type=jnp.float32) + bd_ref[...]
    )


_dec_call = pl.pallas_call(
    _dec_body,
    out_shape=(
        jax.ShapeDtypeStruct((N, D_HID), jnp.float32),
        jax.ShapeDtypeStruct((N, D_IN), jnp.float32),
    ),
)


def kernel(x, edge_index, W_enc, b_enc, W_dec, b_dec):
    src_p = edge_index[0].astype(jnp.int32).reshape(NC, NS, CHUNKS, K)
    dst_p = edge_index[1].astype(jnp.int32).reshape(NC, NS, CHUNKS, K)

    sc_count, sc_aggregate = _sc_kernels()
    counts = sc_count(dst_p)
    h, g, dis = _enc_call(x, W_enc, counts[0, :N], counts[1, :N])
    agg = sc_aggregate(src_p, dst_p, g)
    z, xr = _dec_call(
        agg[0, :N],
        agg[1, :N],
        g,
        dis,
        b_enc.reshape(1, D_HID),
        W_dec,
        b_dec.reshape(1, D_IN),
    )
    return (z, xr)


# speedup vs baseline: 47.9613x; 1.0577x over previous
import functools

import jax
import jax.numpy as jnp
from jax import lax
from jax.experimental import pallas as pl
from jax.experimental.pallas import tpu as pltpu
from jax.experimental.pallas import tpu_sc as plsc

N = 10000
D_IN = 128
D_HID = 64
E = 320000

NC = 2
NS = 16
K = 125
CHUNKS = 80
NPAD = 10240
STRIPE = NPAD // NS
ZB = 80
CW = 16


def _sc_count_body(dst_hbm, out_hbm, dst_v, ones_v, acc_sh, sem):
    c = lax.axis_index("c")
    s = lax.axis_index("s")
    base = s * STRIPE

    def _zero_row(i, carry):
        ones_v[i, :] = jnp.zeros((CW,), jnp.float32)
        return carry

    lax.fori_loop(0, K, _zero_row, 0)

    def _zero_stripe(i, carry):
        pltpu.sync_copy(ones_v.at[pl.ds(0, ZB)], acc_sh.at[pl.ds(base + i * ZB, ZB)])
        return carry

    lax.fori_loop(0, STRIPE // ZB, _zero_stripe, 0)

    def _fill_row(i, carry):
        ones_v[i, :] = jnp.ones((CW,), jnp.float32)
        return carry

    lax.fori_loop(0, K, _fill_row, 0)

    pltpu.sync_copy(dst_hbm.at[c, s], dst_v)
    plsc.subcore_barrier()

    def _scatter(j, carry):
        pltpu.async_copy(ones_v, acc_sh.at[dst_v.at[j]], sem, add=True)
        return carry

    lax.fori_loop(0, CHUNKS, _scatter, 0)

    def _drain(j, carry):
        pltpu.make_async_copy(ones_v, acc_sh.at[dst_v.at[0]], sem).wait()
        return carry

    lax.fori_loop(0, CHUNKS, _drain, 0)
    plsc.subcore_barrier()

    pltpu.sync_copy(acc_sh.at[pl.ds(base, STRIPE)], out_hbm.at[c, pl.ds(base, STRIPE)])


NBUF = 4
ROUNDS = CHUNKS // NBUF


def _sc_aggregate_body(src_hbm, dst_hbm, g_hbm, out_hbm, src_v, dst_v, *rest):
    rows = rest[:NBUF]
    acc_sh = rest[NBUF]
    sem_g = rest[NBUF + 1 : NBUF + 1 + NBUF]
    sem_s = rest[NBUF + 1 + NBUF :]
    c = lax.axis_index("c")
    s = lax.axis_index("s")
    base = s * STRIPE

    def _zero_row(i, carry):
        for kk in range(D_HID // 16):
            rows[0][i, pl.ds(kk * 16, 16)] = jnp.zeros((16,), jnp.float32)
        return carry

    lax.fori_loop(0, K, _zero_row, 0)

    def _zero_stripe(i, carry):
        pltpu.sync_copy(rows[0].at[pl.ds(0, ZB)], acc_sh.at[pl.ds(base + i * ZB, ZB)])
        return carry

    lax.fori_loop(0, STRIPE // ZB, _zero_stripe, 0)

    pltpu.sync_copy(src_hbm.at[c, s], src_v)
    pltpu.sync_copy(dst_hbm.at[c, s], dst_v)
    plsc.subcore_barrier()

    for b in range(NBUF):
        pltpu.async_copy(g_hbm.at[src_v.at[b]], rows[b], sem_g[b])

    def _round(r, carry):
        j0 = r * NBUF
        for b in range(NBUF):
            pltpu.make_async_copy(g_hbm.at[src_v.at[j0 + b]], rows[b], sem_g[b]).wait()
            pltpu.sync_copy(rows[b], acc_sh.at[dst_v.at[j0 + b]], add=True)

            @pl.when(r < ROUNDS - 1)
            def _():
                pltpu.async_copy(g_hbm.at[src_v.at[j0 + NBUF + b]], rows[b], sem_g[b])

        return carry

    lax.fori_loop(0, ROUNDS, _round, 0)
    plsc.subcore_barrier()

    pltpu.sync_copy(acc_sh.at[pl.ds(base, STRIPE)], out_hbm.at[c, pl.ds(base, STRIPE)])


@functools.lru_cache(maxsize=None)
def _sc_kernels():
    mesh = plsc.VectorSubcoreMesh(
        core_axis_name="c", subcore_axis_name="s", num_cores=NC, num_subcores=NS
    )
    count = pl.kernel(
        _sc_count_body,
        out_type=jax.ShapeDtypeStruct((NC, NPAD, CW), jnp.float32),
        mesh=mesh,
        scratch_types=[
            pltpu.VMEM((CHUNKS, K), jnp.int32),
            pltpu.VMEM((K, CW), jnp.float32),
            pltpu.VMEM_SHARED((NPAD, CW), jnp.float32),
            pltpu.SemaphoreType.DMA,
        ],
        name="sc_degree_histogram",
    )
    aggregate = pl.kernel(
        _sc_aggregate_body,
        out_type=jax.ShapeDtypeStruct((NC, NPAD, D_HID), jnp.float32),
        mesh=mesh,
        scratch_types=[
            pltpu.VMEM((CHUNKS, K), jnp.int32),
            pltpu.VMEM((CHUNKS, K), jnp.int32),
        ]
        + [pltpu.VMEM((K, D_HID), jnp.float32) for _ in range(NBUF)]
        + [pltpu.VMEM_SHARED((NPAD, D_HID), jnp.float32)]
        + [pltpu.SemaphoreType.DMA for _ in range(2 * NBUF)],
        name="sc_edge_aggregate",
        compiler_params=pltpu.CompilerParams(use_tc_tiling_on_sc=False),
    )
    return count, aggregate


def _enc_body(x_ref, w_ref, c0_ref, c1_ref, h_ref, g_ref, dis_ref):
    deg = c0_ref[:, 0:1] + c1_ref[:, 0:1] + 1.0
    dis = lax.rsqrt(deg)
    h = jnp.dot(x_ref[...], w_ref[...], preferred_element_type=jnp.float32)
    h_ref[...] = h
    g_ref[...] = h * dis
    dis_ref[...] = dis


_enc_call = pl.pallas_call(
    _enc_body,
    out_shape=(
        jax.ShapeDtypeStruct((N, D_HID), jnp.float32),
        jax.ShapeDtypeStruct((N, D_HID), jnp.float32),
        jax.ShapeDtypeStruct((N, 1), jnp.float32),
    ),
)


def _dec_body(a0_ref, a1_ref, g_ref, dis_ref, be_ref, wd_ref, bd_ref, z_ref, xr_ref):
    agg = a0_ref[...] + a1_ref[...] + g_ref[...]
    pre = agg * dis_ref[...] + be_ref[...]
    z = jnp.maximum(pre, 0.0)
    z_ref[...] = z
    xr_ref[...] = (
        jnp.dot(z, wd_ref[...], preferred_element_type=jnp.float32) + bd_ref[...]
    )


_dec_call = pl.pallas_call(
    _dec_body,
    out_shape=(
        jax.ShapeDtypeStruct((N, D_HID), jnp.float32),
        jax.ShapeDtypeStruct((N, D_IN), jnp.float32),
    ),
)


def kernel(x, edge_index, W_enc, b_enc, W_dec, b_dec):
    src_p = edge_index[0].astype(jnp.int32).reshape(NC, NS, CHUNKS, K)
    dst_p = edge_index[1].astype(jnp.int32).reshape(NC, NS, CHUNKS, K)

    sc_count, sc_aggregate = _sc_kernels()
    counts = sc_count(dst_p)
    h, g, dis = _enc_call(x, W_enc, counts[0, :N], counts[1, :N])
    agg = sc_aggregate(src_p, dst_p, g)
    z, xr = _dec_call(
        agg[0, :N],
        agg[1, :N],
        g,
        dis,
        b_enc.reshape(1, D_HID),
        W_dec,
        b_dec.reshape(1, D_IN),
    )
    return (z, xr)
